# baseline (speedup 1.0000x reference)
import numpy as np
import jax
import jax.numpy as jnp
from jax.experimental import pallas as pl
from jax.experimental.pallas import tpu as pltpu

NCLS = 32
TRI = NCLS * (NCLS - 1) // 2
HID = 32 * TRI
INF = 512
BATCH = 256

_i_idx, _j_idx = np.triu_indices(NCLS, k=1)
_S_np = np.zeros((2 * TRI, NCLS), np.float32)
_S_np[2 * np.arange(TRI), _i_idx] = 1.0
_S_np[2 * np.arange(TRI) + 1, _j_idx] = 1.0

_W1_CHUNK = 512
_NW1 = HID // _W1_CHUNK
_BN = 256
_NJ = HID // _BN

_BF = jnp.bfloat16
_F32 = jnp.float32


def _body(x_ref, w1_hbm, b1_ref, w2_hbm, b2_ref, wout_hbm, s_ref, bout_ref,
          out_ref, h1_ref, w2_buf, w1_buf, wout_buf, acc_ref,
          w2_sem, w1_sem, wout_sem):
    cp = pltpu.make_async_copy

    _H2 = HID // 2

    def w2_dma_a(j, slot):
        return cp(w2_hbm.at[pl.ds(0, _H2), pl.ds(j * _BN, _BN)],
                  w2_buf.at[slot, pl.ds(0, _H2)], w2_sem.at[slot, 0])

    def w2_dma_b(j, slot):
        return cp(w2_hbm.at[pl.ds(_H2, _H2), pl.ds(j * _BN, _BN)],
                  w2_buf.at[slot, pl.ds(_H2, _H2)], w2_sem.at[slot, 1])

    def wout_dma(j, slot):
        return cp(wout_hbm.at[pl.ds(j * _BN, _BN), :], wout_buf.at[slot],
                  wout_sem.at[slot])

    w2_dma_a(0, 0).start()
    w2_dma_b(0, 0).start()
    wout_dma(0, 0).start()

    def w1_dma(k, slot):
        return cp(w1_hbm.at[:, pl.ds(k * _W1_CHUNK, _W1_CHUNK)],
                  w1_buf.at[slot], w1_sem.at[slot])

    xb = x_ref[...].astype(_BF)
    w1_dma(0, 0).start()
    for k in range(_NW1):
        if k + 1 < _NW1:
            w1_dma(k + 1, (k + 1) % 2).start()
        w1_dma(k, k % 2).wait()
        h = jnp.dot(xb, w1_buf[k % 2].astype(_BF),
                    preferred_element_type=_F32) + b1_ref[k]
        h1_ref[:, k * _W1_CHUNK:(k + 1) * _W1_CHUNK] = (
            jnp.maximum(h, 0.0).astype(_BF))

    acc_ref[...] = jnp.zeros_like(acc_ref)

    def loop_body(j, carry):
        slot = jax.lax.rem(j, 2)
        nxt = jax.lax.rem(j + 1, 2)

        @pl.when(j + 1 < _NJ)
        def _():
            w2_dma_a(j + 1, nxt).start()
            w2_dma_b(j + 1, nxt).start()
            wout_dma(j + 1, nxt).start()

        w2_dma_a(j, slot).wait()
        w2_dma_b(j, slot).wait()

        h2 = jnp.dot(h1_ref[...], w2_buf[slot].astype(_BF),
                     preferred_element_type=_F32)
        h2 = jnp.maximum(h2 + b2_ref[j], 0.0)
        wout_dma(j, slot).wait()
        acc_ref[...] += jnp.dot(h2.astype(_BF), wout_buf[slot].astype(_BF),
                                preferred_element_type=_F32)
        return carry

    jax.lax.fori_loop(0, _NJ, loop_body, 0)

    sb = s_ref[...].astype(_BF)
    votes = jnp.dot(acc_ref[...].astype(_BF), sb, preferred_element_type=_F32)
    bias = jnp.dot(bout_ref[...].astype(_BF), sb, preferred_element_type=_F32)
    out_ref[...] = votes + bias


def kernel(x, W1, b1, W2, b2, Wout, bout):
    return pl.pallas_call(
        _body,
        out_shape=jax.ShapeDtypeStruct((BATCH, NCLS), _F32),
        in_specs=[
            pl.BlockSpec(memory_space=pltpu.VMEM),
            pl.BlockSpec(memory_space=pl.ANY),
            pl.BlockSpec(memory_space=pltpu.VMEM),
            pl.BlockSpec(memory_space=pl.ANY),
            pl.BlockSpec(memory_space=pltpu.VMEM),
            pl.BlockSpec(memory_space=pl.ANY),
            pl.BlockSpec(memory_space=pltpu.VMEM),
            pl.BlockSpec(memory_space=pltpu.VMEM),
        ],
        out_specs=pl.BlockSpec(memory_space=pltpu.VMEM),
        scratch_shapes=[
            pltpu.VMEM((BATCH, HID), _BF),
            pltpu.VMEM((2, HID, _BN), _F32),
            pltpu.VMEM((2, INF, _W1_CHUNK), _F32),
            pltpu.VMEM((2, _BN, 2 * TRI), _F32),
            pltpu.VMEM((BATCH, 2 * TRI), _F32),
            pltpu.SemaphoreType.DMA((2, 2)),
            pltpu.SemaphoreType.DMA((2,)),
            pltpu.SemaphoreType.DMA((2,)),
        ],
        compiler_params=pltpu.CompilerParams(
            vmem_limit_bytes=56 * 1024 * 1024,
        ),
        name="bsq_fused",
    )(x, W1, b1.reshape(_NW1, 1, _W1_CHUNK), W2,
      b2.reshape(_NJ, 1, _BN), Wout, jnp.asarray(_S_np),
      bout.reshape(1, 2 * TRI))


# speedup vs baseline: 1.0116x; 1.0116x over previous
import numpy as np
import jax
import jax.numpy as jnp
from jax.experimental import pallas as pl
from jax.experimental.pallas import tpu as pltpu

NCLS = 32
TRI = NCLS * (NCLS - 1) // 2
HID = 32 * TRI
INF = 512
BATCH = 256

_i_idx, _j_idx = np.triu_indices(NCLS, k=1)
_S_np = np.zeros((2 * TRI, NCLS), np.float32)
_S_np[2 * np.arange(TRI), _i_idx] = 1.0
_S_np[2 * np.arange(TRI) + 1, _j_idx] = 1.0

_W1_CHUNK = 3968
_NW1 = HID // _W1_CHUNK
_BN = 256
_NJ = HID // _BN

_BF = jnp.bfloat16
_F32 = jnp.float32


def _body(x_ref, w1_hbm, b1_ref, w2_hbm, b2_ref, wout_hbm, s_ref, bout_ref,
          out_ref, h1_ref, w2_buf, w1_buf, wout_buf, acc_ref,
          w2_sem, w1_sem, wout_sem):
    cp = pltpu.make_async_copy

    _H2 = HID // 2

    def w2_dma_a(j, slot):
        return cp(w2_hbm.at[pl.ds(0, _H2), pl.ds(j * _BN, _BN)],
                  w2_buf.at[slot, pl.ds(0, _H2)], w2_sem.at[slot, 0])

    def w2_dma_b(j, slot):
        return cp(w2_hbm.at[pl.ds(_H2, _H2), pl.ds(j * _BN, _BN)],
                  w2_buf.at[slot, pl.ds(_H2, _H2)], w2_sem.at[slot, 1])

    def wout_dma(j, slot):
        return cp(wout_hbm.at[pl.ds(j * _BN, _BN), :], wout_buf.at[slot],
                  wout_sem.at[slot])

    w2_dma_a(0, 0).start()
    w2_dma_b(0, 0).start()
    wout_dma(0, 0).start()

    xb = x_ref[...].astype(_BF)
    for k in range(_NW1):
        dma = cp(w1_hbm.at[:, pl.ds(k * _W1_CHUNK, _W1_CHUNK)],
                 w1_buf.at[0], w1_sem.at[0])
        dma.start()
        dma.wait()
        h = jnp.dot(xb, w1_buf[0].astype(_BF),
                    preferred_element_type=_F32) + b1_ref[k]
        h1_ref[:, k * _W1_CHUNK:(k + 1) * _W1_CHUNK] = (
            jnp.maximum(h, 0.0).astype(_BF))

    acc_ref[...] = jnp.zeros_like(acc_ref)

    def loop_body(j, carry):
        slot = jax.lax.rem(j, 2)
        nxt = jax.lax.rem(j + 1, 2)

        @pl.when(j + 1 < _NJ)
        def _():
            w2_dma_a(j + 1, nxt).start()
            w2_dma_b(j + 1, nxt).start()
            wout_dma(j + 1, nxt).start()

        w2_dma_a(j, slot).wait()
        w2_dma_b(j, slot).wait()

        h2 = jnp.dot(h1_ref[...], w2_buf[slot].astype(_BF),
                     preferred_element_type=_F32)
        h2 = jnp.maximum(h2 + b2_ref[j], 0.0)
        wout_dma(j, slot).wait()
        acc_ref[...] += jnp.dot(h2.astype(_BF), wout_buf[slot].astype(_BF),
                                preferred_element_type=_F32)
        return carry

    jax.lax.fori_loop(0, _NJ, loop_body, 0)

    sb = s_ref[...].astype(_BF)
    votes = jnp.dot(acc_ref[...].astype(_BF), sb, preferred_element_type=_F32)
    bias = jnp.dot(bout_ref[...].astype(_BF), sb, preferred_element_type=_F32)
    out_ref[...] = votes + bias


def kernel(x, W1, b1, W2, b2, Wout, bout):
    return pl.pallas_call(
        _body,
        out_shape=jax.ShapeDtypeStruct((BATCH, NCLS), _F32),
        in_specs=[
            pl.BlockSpec(memory_space=pltpu.VMEM),
            pl.BlockSpec(memory_space=pl.ANY),
            pl.BlockSpec(memory_space=pltpu.VMEM),
            pl.BlockSpec(memory_space=pl.ANY),
            pl.BlockSpec(memory_space=pltpu.VMEM),
            pl.BlockSpec(memory_space=pl.ANY),
            pl.BlockSpec(memory_space=pltpu.VMEM),
            pl.BlockSpec(memory_space=pltpu.VMEM),
        ],
        out_specs=pl.BlockSpec(memory_space=pltpu.VMEM),
        scratch_shapes=[
            pltpu.VMEM((BATCH, HID), _BF),
            pltpu.VMEM((2, HID, _BN), _F32),
            pltpu.VMEM((1, INF, _W1_CHUNK), _F32),
            pltpu.VMEM((2, _BN, 2 * TRI), _F32),
            pltpu.VMEM((BATCH, 2 * TRI), _F32),
            pltpu.SemaphoreType.DMA((2, 2)),
            pltpu.SemaphoreType.DMA((1,)),
            pltpu.SemaphoreType.DMA((2,)),
        ],
        compiler_params=pltpu.CompilerParams(
            vmem_limit_bytes=56 * 1024 * 1024,
        ),
        name="bsq_fused",
    )(x, W1, b1.reshape(_NW1, 1, _W1_CHUNK), W2,
      b2.reshape(_NJ, 1, _BN), Wout, jnp.asarray(_S_np),
      bout.reshape(1, 2 * TRI))
